# Initial kernel scaffold; baseline (speedup 1.0000x reference)
import jax
import jax.numpy as jnp
from jax.experimental import pallas as pl


def kernel(edge_attr, W0, W1, W2):
    raise NotImplementedError("write your pallas kernel here")


# speedup vs baseline: 1.2650x; 1.2650x over previous
import functools

import jax
import jax.numpy as jnp
from jax import lax
from jax.experimental import pallas as pl
from jax.experimental.pallas import tpu as pltpu
from jax.experimental.pallas import tpu_sc as plsc

E = 800000
D = 64
NC = 2
NS = 16
NW = NC * NS
L = 16
C = 128
NCHUNK = E // C
KMAX = (NCHUNK + NW - 1) // NW

_mesh = plsc.VectorSubcoreMesh(core_axis_name="c", subcore_axis_name="s")


@functools.partial(
    pl.kernel,
    out_type=jax.ShapeDtypeStruct((E, D), jnp.float32),
    mesh=_mesh,
    compiler_params=pltpu.CompilerParams(use_tc_tiling_on_sc=False),
    scratch_types=[
        pltpu.VMEM((C,), jnp.int32),
        pltpu.VMEM((C,), jnp.int32),
        pltpu.VMEM((C,), jnp.int32),
        pltpu.VMEM((C,), jnp.int32),
        pltpu.VMEM((C, D), jnp.float32),
        pltpu.SemaphoreType.DMA,
    ],
)
def _sc_lookup(a0_h, a1_h, a2_h, tab_h, out_h, a0_v, a1_v, a2_v, idx_v,
               rows_v, sem):
    wid = lax.axis_index("s") * NC + lax.axis_index("c")

    def step(k, carry):
        chunk = k * NW + wid

        @pl.when(chunk < NCHUNK)
        def _():
            ebase = chunk * C
            pltpu.sync_copy(a0_h.at[pl.ds(ebase, C)], a0_v)
            pltpu.sync_copy(a1_h.at[pl.ds(ebase, C)], a1_v)
            pltpu.sync_copy(a2_h.at[pl.ds(ebase, C)], a2_v)
            for g in range(C // L):
                s = pl.ds(g * L, L)
                idx_v[s] = a0_v[s] * 12 + a1_v[s] * 2 + a2_v[s]
            pltpu.async_copy(tab_h.at[idx_v], rows_v, sem).wait()
            pltpu.sync_copy(rows_v, out_h.at[pl.ds(ebase, C)])

        return carry

    lax.fori_loop(0, KMAX, step, 0)


def kernel(edge_attr, W0, W1, W2):
    ea = edge_attr.astype(jnp.int32)
    a0 = ea[:, 0]
    a1 = ea[:, 1]
    a2 = ea[:, 2]
    tab = (W0[:, None, None, :] + W1[None, :, None, :]
           + W2[None, None, :, :]).reshape(60, D)
    return _sc_lookup(a0, a1, a2, tab)
